# Initial kernel scaffold; baseline (speedup 1.0000x reference)
import jax
import jax.numpy as jnp
from jax.experimental import pallas as pl


def kernel(x, edge_index, i, W, b, Wd, bd):
    raise NotImplementedError("write your pallas kernel here")


# speedup vs baseline: 83.8291x; 83.8291x over previous
import functools

import jax
import jax.numpy as jnp
from jax import lax
from jax.experimental import pallas as pl
from jax.experimental.pallas import tpu as pltpu
from jax.experimental.pallas import tpu_sc as plsc

N = 10000
E = 320000
G = 16
D = 128
NPAD = 10240
NT = 16
NC = 2
CH = 80
VR = CH // 16

E_DEG = E // NT
DEG_ROWS = E_DEG // CH
E_S = E // (NC * NT)
S_ROWS = E_S // CH
CHUNK = NPAD // NT
SELF_ROWS = CHUNK // CH

KDEG = 5
KS = 5


def _rsqrt_sc(d):
    y = lax.bitcast_convert_type(
        jnp.int32(0x5F3759DF) - (lax.bitcast_convert_type(d, jnp.int32) >> 1),
        jnp.float32)
    for _ in range(3):
        y = y * (1.5 - 0.5 * d * y * y)
    return y


def _sc_body(src1, dst1, i_hbm, zeros_hbm, ones_hbm, out_s,
             deg_sh, dinv_sh, s_sh,
             zerobuf, onesrow, srcbuf, dstbuf_s,
             i_priv, dinv_priv, workbuf, idxbuf, valbuf, selfidx, selfval,
             sem):
    c = lax.axis_index("c")
    t = lax.axis_index("s")

    pltpu.sync_copy(zeros_hbm, zerobuf)
    pltpu.sync_copy(zerobuf.at[pl.ds(0, CHUNK)], deg_sh.at[pl.ds(t * CHUNK, CHUNK)])
    pltpu.sync_copy(zerobuf, s_sh.at[pl.ds(t * N, N)])
    pltpu.sync_copy(ones_hbm, onesrow)
    plsc.subcore_barrier()

    for h in range(2):
        pltpu.sync_copy(dst1.at[pl.ds(t * E_DEG + h * E_S, E_S)], dstbuf_s)

        def deg_fill(r, carry):
            for k in range(VR):
                idxbuf[r, pl.ds(k * 16, 16)] = dstbuf_s[pl.ds(r * CH + k * 16, 16)]
            return carry

        lax.fori_loop(0, S_ROWS, deg_fill, None)

        def deg_group(gi, carry):
            descs = [
                pltpu.async_copy(onesrow, deg_sh.at[idxbuf.at[gi * KDEG + j]],
                                 sem, add=True)
                for j in range(KDEG)
            ]
            for dsc in descs:
                dsc.wait()
            return carry

        lax.fori_loop(0, S_ROWS // KDEG, deg_group, None)
    plsc.subcore_barrier()

    pltpu.sync_copy(deg_sh.at[pl.ds(t * CHUNK, CHUNK)], workbuf)

    def dinv_step(j, carry):
        d = workbuf[pl.ds(j * 16, 16)] + 1.0
        workbuf[pl.ds(j * 16, 16)] = _rsqrt_sc(d)
        return carry

    lax.fori_loop(0, CHUNK // 16, dinv_step, None)
    pltpu.sync_copy(workbuf, dinv_sh.at[pl.ds(t * CHUNK, CHUNK)])
    plsc.subcore_barrier()

    pltpu.sync_copy(dinv_sh, dinv_priv)
    pltpu.sync_copy(i_hbm, i_priv)
    w = c * NT + t
    pltpu.sync_copy(src1.at[pl.ds(w * E_S, E_S)], srcbuf)
    pltpu.sync_copy(dst1.at[pl.ds(w * E_S, E_S)], dstbuf_s)

    def edge_step(r, carry):
        for k in range(VR):
            sl = pl.ds(k * 16, 16)
            sv = srcbuf[pl.ds(r * CH + k * 16, 16)]
            dv = dstbuf_s[pl.ds(r * CH + k * 16, 16)]
            g = plsc.load_gather(i_priv, [dv])
            da = plsc.load_gather(dinv_priv, [sv])
            db = plsc.load_gather(dinv_priv, [dv])
            idxbuf[r, sl] = sv * G + g
            valbuf[r, sl] = da * db
        return carry

    lax.fori_loop(0, S_ROWS, edge_step, None)

    def s_group(gi, carry):
        descs = [
            pltpu.async_copy(valbuf.at[gi * KS + j], s_sh.at[idxbuf.at[gi * KS + j]],
                             sem, add=True)
            for j in range(KS)
        ]
        for dsc in descs:
            dsc.wait()
        return carry

    lax.fori_loop(0, S_ROWS // KS, s_group, None)

    @pl.when(c == 0)
    def _self_loops():
        def self_step(r, carry):
            for k in range(VR):
                sl = pl.ds(k * 16, 16)
                v = t * CHUNK + r * CH + k * 16 + lax.iota(jnp.int32, 16)
                valid = v < N
                vc = jnp.minimum(v, N - 1)
                g = plsc.load_gather(i_priv, [vc])
                dv = plsc.load_gather(dinv_priv, [vc])
                selfidx[r, sl] = jnp.where(valid, vc * G + g, 0)
                selfval[r, sl] = jnp.where(valid, dv * dv, 0.0)
            return carry

        lax.fori_loop(0, SELF_ROWS, self_step, None)

        def self_group(gi, carry):
            descs = [
                pltpu.async_copy(selfval.at[j], s_sh.at[selfidx.at[j]],
                                 sem, add=True)
                for j in range(SELF_ROWS)
            ]
            for dsc in descs:
                dsc.wait()
            return carry

        lax.fori_loop(0, 1, self_group, None)

    plsc.subcore_barrier()

    pltpu.sync_copy(s_sh.at[pl.ds(t * N, N)], zerobuf)
    pltpu.sync_copy(zerobuf, out_s.at[pl.ds(w * N, N)])


_sc_scatter = functools.partial(
    pl.kernel,
    out_type=jax.ShapeDtypeStruct((NC * NT * N,), jnp.float32),
    mesh=plsc.VectorSubcoreMesh(core_axis_name="c", subcore_axis_name="s"),
    compiler_params=pltpu.CompilerParams(needs_layout_passes=False),
    scratch_types=[
        pltpu.VMEM_SHARED((NPAD,), jnp.float32),
        pltpu.VMEM_SHARED((NPAD,), jnp.float32),
        pltpu.VMEM_SHARED((N * G,), jnp.float32),
        pltpu.VMEM((N,), jnp.float32),
        pltpu.VMEM((CH,), jnp.float32),
        pltpu.VMEM((E_S,), jnp.int32),
        pltpu.VMEM((E_S,), jnp.int32),
        pltpu.VMEM((N,), jnp.int32),
        pltpu.VMEM((NPAD,), jnp.float32),
        pltpu.VMEM((CHUNK,), jnp.float32),
        pltpu.VMEM((S_ROWS, CH), jnp.int32),
        pltpu.VMEM((S_ROWS, CH), jnp.float32),
        pltpu.VMEM((SELF_ROWS, CH), jnp.int32),
        pltpu.VMEM((SELF_ROWS, CH), jnp.float32),
        pltpu.SemaphoreType.DMA,
    ],
)(_sc_body)


def _tc_body(s_ref, x_ref, i_ref, w_ref, b_ref, wd_ref, bd_ref, o_ref):
    S = s_ref[0] + s_ref[1]
    X = x_ref[...]
    H = jnp.dot(X, w_ref[...])
    P = lax.dot_general(S, H, (((0,), (0,)), ((), ())),
                        preferred_element_type=jnp.float32,
                        precision=lax.Precision.HIGHEST)
    giota = lax.broadcasted_iota(jnp.int32, (N, G), 1)
    onehot = jnp.where(i_ref[...] == giota, 1.0, 0.0)
    ncol = lax.dot_general(onehot, jnp.ones((N, 1), jnp.float32),
                           (((0,), (0,)), ((), ())),
                           precision=lax.Precision.HIGHEST)
    pooled = P + ncol * b_ref[...]
    logits = jnp.dot(pooled, wd_ref[...],
                     precision=lax.Precision.HIGHEST) + bd_ref[...]
    m = jnp.max(logits, axis=1, keepdims=True)
    e = jnp.exp(logits - m)
    o_ref[...] = e / jnp.sum(e, axis=1, keepdims=True)


def kernel(x, edge_index, i, W, b, Wd, bd):
    src = edge_index[0].astype(jnp.int32)
    dst = edge_index[1].astype(jnp.int32)
    ii = i.astype(jnp.int32)
    zeros_hbm = jnp.zeros((N,), jnp.float32)
    ones_hbm = jnp.ones((CH,), jnp.float32)

    s_flat = _sc_scatter(src, dst, ii, zeros_hbm, ones_hbm)
    s2 = s_flat.reshape(NC, N, G)

    out = pl.pallas_call(
        _tc_body,
        out_shape=jax.ShapeDtypeStruct((G, 10), jnp.float32),
    )(s2, x, ii.reshape(N, 1), W, b.reshape(1, D), Wd, bd.reshape(1, 10))
    return out


# speedup vs baseline: 88.6185x; 1.0571x over previous
import functools

import jax
import jax.numpy as jnp
from jax import lax
from jax.experimental import pallas as pl
from jax.experimental.pallas import tpu as pltpu
from jax.experimental.pallas import tpu_sc as plsc

N = 10000
E = 320000
G = 16
D = 128
NPAD = 10240
NT = 16
NC = 2
CH = 2000
VR = CH // 16

E_DEG = E // NT
E_S = E // (NC * NT)
S_ROWS = E_S // CH
CHUNK = NPAD // NT
SELF_VR = CHUNK // 16


def _rsqrt_sc(d):
    y = lax.bitcast_convert_type(
        jnp.int32(0x5F3759DF) - (lax.bitcast_convert_type(d, jnp.int32) >> 1),
        jnp.float32)
    for _ in range(3):
        y = y * (1.5 - 0.5 * d * y * y)
    return y


def _sc_body(src1, dst1, i_hbm, zeros_hbm, ones_hbm, out_s,
             deg_sh, dinv_sh, s_sh,
             zerobuf, onesrow, srcbuf, dstbuf_s,
             i_priv, dinv_priv, workbuf, idxbuf, valbuf, selfidx, selfval,
             sem):
    c = lax.axis_index("c")
    t = lax.axis_index("s")

    pltpu.sync_copy(zeros_hbm, zerobuf)
    pltpu.sync_copy(zerobuf.at[pl.ds(0, CHUNK)], deg_sh.at[pl.ds(t * CHUNK, CHUNK)])
    pltpu.sync_copy(zerobuf, s_sh.at[pl.ds(t * N, N)])
    pltpu.sync_copy(ones_hbm, onesrow)
    plsc.subcore_barrier()

    for h in range(2):
        pltpu.sync_copy(dst1.at[pl.ds(t * E_DEG + h * E_S, E_S)], dstbuf_s)
        for r in range(S_ROWS):
            def deg_fill(k, carry):
                idxbuf[r, 0, pl.ds(k * 16, 16)] = dstbuf_s[pl.ds(r * CH + k * 16, 16)]
                return carry
            lax.fori_loop(0, VR, deg_fill, None)
        descs = [
            pltpu.async_copy(onesrow, deg_sh.at[idxbuf.at[j, 0]], sem, add=True)
            for j in range(S_ROWS)
        ]
        for dsc in descs:
            dsc.wait()
    plsc.subcore_barrier()

    pltpu.sync_copy(deg_sh.at[pl.ds(t * CHUNK, CHUNK)], workbuf)

    def dinv_step(j, carry):
        d = workbuf[pl.ds(j * 16, 16)] + 1.0
        workbuf[pl.ds(j * 16, 16)] = _rsqrt_sc(d)
        return carry

    lax.fori_loop(0, CHUNK // 16, dinv_step, None)
    pltpu.sync_copy(workbuf, dinv_sh.at[pl.ds(t * CHUNK, CHUNK)])
    plsc.subcore_barrier()

    pltpu.sync_copy(dinv_sh, dinv_priv)
    pltpu.sync_copy(i_hbm, i_priv)
    w = c * NT + t
    pltpu.sync_copy(src1.at[pl.ds(w * E_S, E_S)], srcbuf)
    pltpu.sync_copy(dst1.at[pl.ds(w * E_S, E_S)], dstbuf_s)

    for r in range(S_ROWS):
        def edge_step(k, carry):
            sl = pl.ds(k * 16, 16)
            sv = srcbuf[pl.ds(r * CH + k * 16, 16)]
            dv = dstbuf_s[pl.ds(r * CH + k * 16, 16)]
            g = plsc.load_gather(i_priv, [dv])
            da = plsc.load_gather(dinv_priv, [sv])
            db = plsc.load_gather(dinv_priv, [dv])
            idxbuf[r, 0, sl] = sv * G + g
            valbuf[r, 0, sl] = da * db
            return carry
        lax.fori_loop(0, VR, edge_step, None)

    descs = [
        pltpu.async_copy(valbuf.at[j, 0], s_sh.at[idxbuf.at[j, 0]], sem, add=True)
        for j in range(S_ROWS)
    ]
    for dsc in descs:
        dsc.wait()

    @pl.when(c == 0)
    def _self_loops():
        def self_step(k, carry):
            sl = pl.ds(k * 16, 16)
            v = t * CHUNK + k * 16 + lax.iota(jnp.int32, 16)
            valid = v < N
            vc = jnp.minimum(v, N - 1)
            g = plsc.load_gather(i_priv, [vc])
            dv = plsc.load_gather(dinv_priv, [vc])
            selfidx[0, 0, sl] = jnp.where(valid, vc * G + g, 0)
            selfval[0, 0, sl] = jnp.where(valid, dv * dv, 0.0)
            return carry

        lax.fori_loop(0, SELF_VR, self_step, None)
        pltpu.async_copy(selfval.at[0, 0], s_sh.at[selfidx.at[0, 0]], sem,
                         add=True).wait()

    plsc.subcore_barrier()

    pltpu.sync_copy(s_sh.at[pl.ds(t * N, N)], zerobuf)
    pltpu.sync_copy(zerobuf, out_s.at[pl.ds(w * N, N)])


_sc_scatter = functools.partial(
    pl.kernel,
    out_type=jax.ShapeDtypeStruct((NC * NT * N,), jnp.float32),
    mesh=plsc.VectorSubcoreMesh(core_axis_name="c", subcore_axis_name="s"),
    compiler_params=pltpu.CompilerParams(needs_layout_passes=False),
    scratch_types=[
        pltpu.VMEM_SHARED((NPAD,), jnp.float32),
        pltpu.VMEM_SHARED((NPAD,), jnp.float32),
        pltpu.VMEM_SHARED((N * G,), jnp.float32),
        pltpu.VMEM((N,), jnp.float32),
        pltpu.VMEM((CH,), jnp.float32),
        pltpu.VMEM((E_S,), jnp.int32),
        pltpu.VMEM((E_S,), jnp.int32),
        pltpu.VMEM((N,), jnp.int32),
        pltpu.VMEM((NPAD,), jnp.float32),
        pltpu.VMEM((CHUNK,), jnp.float32),
        pltpu.VMEM((S_ROWS, 1, CH), jnp.int32),
        pltpu.VMEM((S_ROWS, 1, CH), jnp.float32),
        pltpu.VMEM((1, 1, CHUNK), jnp.int32),
        pltpu.VMEM((1, 1, CHUNK), jnp.float32),
        pltpu.SemaphoreType.DMA,
    ],
)(_sc_body)


def _tc_body(s_ref, x_ref, i_ref, w_ref, b_ref, wd_ref, bd_ref, o_ref):
    S = s_ref[0] + s_ref[1]
    X = x_ref[...]
    H = jnp.dot(X, w_ref[...])
    P = lax.dot_general(S, H, (((0,), (0,)), ((), ())),
                        preferred_element_type=jnp.float32,
                        precision=lax.Precision.HIGHEST)
    giota = lax.broadcasted_iota(jnp.int32, (N, G), 1)
    onehot = jnp.where(i_ref[...] == giota, 1.0, 0.0)
    ncol = lax.dot_general(onehot, jnp.ones((N, 1), jnp.float32),
                           (((0,), (0,)), ((), ())),
                           precision=lax.Precision.HIGHEST)
    pooled = P + ncol * b_ref[...]
    logits = jnp.dot(pooled, wd_ref[...],
                     precision=lax.Precision.HIGHEST) + bd_ref[...]
    m = jnp.max(logits, axis=1, keepdims=True)
    e = jnp.exp(logits - m)
    o_ref[...] = e / jnp.sum(e, axis=1, keepdims=True)


def kernel(x, edge_index, i, W, b, Wd, bd):
    src = edge_index[0].astype(jnp.int32)
    dst = edge_index[1].astype(jnp.int32)
    ii = i.astype(jnp.int32)
    zeros_hbm = jnp.zeros((N,), jnp.float32)
    ones_hbm = jnp.ones((CH,), jnp.float32)

    s_flat = _sc_scatter(src, dst, ii, zeros_hbm, ones_hbm)
    s2 = s_flat.reshape(NC, N, G)

    out = pl.pallas_call(
        _tc_body,
        out_shape=jax.ShapeDtypeStruct((G, 10), jnp.float32),
    )(s2, x, ii.reshape(N, 1), W, b.reshape(1, D), Wd, bd.reshape(1, 10))
    return out


# speedup vs baseline: 94.7694x; 1.0694x over previous
import functools

import jax
import jax.numpy as jnp
from jax import lax
from jax.experimental import pallas as pl
from jax.experimental.pallas import tpu as pltpu
from jax.experimental.pallas import tpu_sc as plsc

N = 10000
E = 320000
G = 16
D = 128
NPAD = 10240
NT = 16
NC = 2
CH = 2000
VR = CH // 16
UNROLL = 5

DEG_ROWS = E // NT // CH
S_ROWS = E // (NC * NT) // CH
CHUNK = NPAD // NT
SELF_VR = CHUNK // 16


def _rsqrt_sc(d):
    y = lax.bitcast_convert_type(
        jnp.int32(0x5F3759DF) - (lax.bitcast_convert_type(d, jnp.int32) >> 1),
        jnp.float32)
    for _ in range(3):
        y = y * (1.5 - 0.5 * d * y * y)
    return y


def _sc_body(src3, dst3, i_hbm, zeros_hbm, ones_hbm, out_s,
             deg_sh, dinv_sh, s_sh,
             zerobuf, onesrow, degbuf, srcbuf, dstbuf,
             i_priv, dinv_priv, workbuf, idxbuf, valbuf, selfidx, selfval,
             sem_in, sem_z, sem_sc):
    c = lax.axis_index("c")
    t = lax.axis_index("s")
    w = c * NT + t

    z0 = pltpu.async_copy(zeros_hbm, zerobuf, sem_z)
    loads = [
        pltpu.async_copy(ones_hbm, onesrow, sem_in),
        pltpu.async_copy(i_hbm, i_priv, sem_in),
        pltpu.async_copy(dst3.at[pl.ds(t * DEG_ROWS, DEG_ROWS)], degbuf, sem_in),
        pltpu.async_copy(src3.at[pl.ds(w * S_ROWS, S_ROWS)], srcbuf, sem_in),
        pltpu.async_copy(dst3.at[pl.ds(w * S_ROWS, S_ROWS)], dstbuf, sem_in),
    ]
    z0.wait()
    zs = [
        pltpu.async_copy(zerobuf.at[pl.ds(0, CHUNK)],
                         deg_sh.at[pl.ds(t * CHUNK, CHUNK)], sem_z),
        pltpu.async_copy(zerobuf, s_sh.at[pl.ds(t * N, N)], sem_z),
    ]
    for dsc in loads:
        dsc.wait()
    for dsc in zs:
        dsc.wait()
    plsc.subcore_barrier()

    deg_descs = [
        pltpu.async_copy(onesrow, deg_sh.at[degbuf.at[j, 0]], sem_sc, add=True)
        for j in range(DEG_ROWS)
    ]

    for r in range(S_ROWS):
        def idx_step(jo, carry):
            for u in range(UNROLL):
                k = jo * UNROLL + u
                sl = pl.ds(k * 16, 16)
                sv = srcbuf[r, 0, sl]
                dv = dstbuf[r, 0, sl]
                g = plsc.load_gather(i_priv, [dv])
                idxbuf[r, 0, sl] = sv * G + g
            return carry
        lax.fori_loop(0, VR // UNROLL, idx_step, None)

    @pl.when(c == 0)
    def _self_idx():
        def self_idx_step(k, carry):
            sl = pl.ds(k * 16, 16)
            v = t * CHUNK + k * 16 + lax.iota(jnp.int32, 16)
            valid = v < N
            vc = jnp.minimum(v, N - 1)
            g = plsc.load_gather(i_priv, [vc])
            selfidx[0, 0, sl] = jnp.where(valid, vc * G + g, 0)
            return carry
        lax.fori_loop(0, SELF_VR, self_idx_step, None)

    for dsc in deg_descs:
        dsc.wait()
    plsc.subcore_barrier()

    pltpu.sync_copy(deg_sh.at[pl.ds(t * CHUNK, CHUNK)], workbuf)

    def dinv_step(j, carry):
        d = workbuf[pl.ds(j * 16, 16)] + 1.0
        workbuf[pl.ds(j * 16, 16)] = _rsqrt_sc(d)
        return carry

    lax.fori_loop(0, CHUNK // 16, dinv_step, None)
    pltpu.sync_copy(workbuf, dinv_sh.at[pl.ds(t * CHUNK, CHUNK)])
    plsc.subcore_barrier()

    pltpu.sync_copy(dinv_sh, dinv_priv)

    for r in range(S_ROWS):
        def val_step(jo, carry):
            for u in range(UNROLL):
                k = jo * UNROLL + u
                sl = pl.ds(k * 16, 16)
                sv = srcbuf[r, 0, sl]
                dv = dstbuf[r, 0, sl]
                da = plsc.load_gather(dinv_priv, [sv])
                db = plsc.load_gather(dinv_priv, [dv])
                valbuf[r, 0, sl] = da * db
            return carry
        lax.fori_loop(0, VR // UNROLL, val_step, None)

    s_descs = [
        pltpu.async_copy(valbuf.at[j, 0], s_sh.at[idxbuf.at[j, 0]], sem_sc,
                         add=True)
        for j in range(S_ROWS)
    ]

    @pl.when(c == 0)
    def _self_loops():
        def self_val_step(k, carry):
            sl = pl.ds(k * 16, 16)
            v = t * CHUNK + k * 16 + lax.iota(jnp.int32, 16)
            valid = v < N
            vc = jnp.minimum(v, N - 1)
            dv = plsc.load_gather(dinv_priv, [vc])
            selfval[0, 0, sl] = jnp.where(valid, dv * dv, 0.0)
            return carry

        lax.fori_loop(0, SELF_VR, self_val_step, None)
        pltpu.async_copy(selfval.at[0, 0], s_sh.at[selfidx.at[0, 0]], sem_sc,
                         add=True).wait()

    for dsc in s_descs:
        dsc.wait()
    plsc.subcore_barrier()

    pltpu.sync_copy(s_sh.at[pl.ds(t * N, N)], zerobuf)
    pltpu.sync_copy(zerobuf, out_s.at[pl.ds(w * N, N)])


_sc_scatter = functools.partial(
    pl.kernel,
    out_type=jax.ShapeDtypeStruct((NC * NT * N,), jnp.float32),
    mesh=plsc.VectorSubcoreMesh(core_axis_name="c", subcore_axis_name="s"),
    compiler_params=pltpu.CompilerParams(needs_layout_passes=False),
    scratch_types=[
        pltpu.VMEM_SHARED((NPAD,), jnp.float32),
        pltpu.VMEM_SHARED((NPAD,), jnp.float32),
        pltpu.VMEM_SHARED((N * G,), jnp.float32),
        pltpu.VMEM((N,), jnp.float32),
        pltpu.VMEM((CH,), jnp.float32),
        pltpu.VMEM((DEG_ROWS, 1, CH), jnp.int32),
        pltpu.VMEM((S_ROWS, 1, CH), jnp.int32),
        pltpu.VMEM((S_ROWS, 1, CH), jnp.int32),
        pltpu.VMEM((N,), jnp.int32),
        pltpu.VMEM((NPAD,), jnp.float32),
        pltpu.VMEM((CHUNK,), jnp.float32),
        pltpu.VMEM((S_ROWS, 1, CH), jnp.int32),
        pltpu.VMEM((S_ROWS, 1, CH), jnp.float32),
        pltpu.VMEM((1, 1, CHUNK), jnp.int32),
        pltpu.VMEM((1, 1, CHUNK), jnp.float32),
        pltpu.SemaphoreType.DMA,
        pltpu.SemaphoreType.DMA,
        pltpu.SemaphoreType.DMA,
    ],
)(_sc_body)


def _tc_body(s_ref, x_ref, i_ref, w_ref, b_ref, wd_ref, bd_ref, o_ref):
    S = s_ref[0] + s_ref[1]
    X = x_ref[...]
    H = jnp.dot(X, w_ref[...])
    P = lax.dot_general(S, H, (((0,), (0,)), ((), ())),
                        preferred_element_type=jnp.float32,
                        precision=lax.Precision.HIGHEST)
    giota = lax.broadcasted_iota(jnp.int32, (N, G), 1)
    onehot = jnp.where(i_ref[...] == giota, 1.0, 0.0)
    ncol = lax.dot_general(onehot, jnp.ones((N, 1), jnp.float32),
                           (((0,), (0,)), ((), ())),
                           precision=lax.Precision.HIGHEST)
    pooled = P + ncol * b_ref[...]
    logits = jnp.dot(pooled, wd_ref[...],
                     precision=lax.Precision.HIGHEST) + bd_ref[...]
    m = jnp.max(logits, axis=1, keepdims=True)
    e = jnp.exp(logits - m)
    o_ref[...] = e / jnp.sum(e, axis=1, keepdims=True)


def kernel(x, edge_index, i, W, b, Wd, bd):
    src = edge_index[0].astype(jnp.int32)
    dst = edge_index[1].astype(jnp.int32)
    src3 = src.reshape(E // CH, 1, CH)
    dst3 = dst.reshape(E // CH, 1, CH)
    ii = i.astype(jnp.int32)
    zeros_hbm = jnp.zeros((N,), jnp.float32)
    ones_hbm = jnp.ones((CH,), jnp.float32)

    s_flat = _sc_scatter(src3, dst3, ii, zeros_hbm, ones_hbm)
    s2 = s_flat.reshape(NC, N, G)

    out = pl.pallas_call(
        _tc_body,
        out_shape=jax.ShapeDtypeStruct((G, 10), jnp.float32),
    )(s2, x, ii.reshape(N, 1), W, b.reshape(1, D), Wd, bd.reshape(1, 10))
    return out


# speedup vs baseline: 125.3569x; 1.3228x over previous
import functools

import jax
import jax.numpy as jnp
from jax import lax
from jax.experimental import pallas as pl
from jax.experimental.pallas import tpu as pltpu
from jax.experimental.pallas import tpu_sc as plsc

N = 10000
E = 320000
G = 16
D = 128
NPAD = 10240
NT = 16
NC = 2
CH = 2000
VR = CH // 16
UNROLL = 5

DEG_ROWS = E // NT // CH
S_ROWS = E // (NC * NT) // CH
CHUNK = NPAD // NT
SELF_VR = CHUNK // 16


def _rsqrt_sc(d):
    y = lax.bitcast_convert_type(
        jnp.int32(0x5F3759DF) - (lax.bitcast_convert_type(d, jnp.int32) >> 1),
        jnp.float32)
    for _ in range(3):
        y = y * (1.5 - 0.5 * d * y * y)
    return y


def _sc_body(ei4, i_hbm, out_s,
             deg_sh, dinv_sh, s_sh,
             zerobuf, onesrow, degbuf, srcbuf, dstbuf,
             i_priv, dinv_priv, workbuf, idxbuf, valbuf, selfidx, selfval,
             sem_in, sem_z, sem_sc):
    c = lax.axis_index("c")
    t = lax.axis_index("s")
    w = c * NT + t

    loads = [
        pltpu.async_copy(i_hbm, i_priv, sem_in),
        pltpu.async_copy(ei4.at[1, pl.ds(t * DEG_ROWS, DEG_ROWS)], degbuf, sem_in),
        pltpu.async_copy(ei4.at[0, pl.ds(w * S_ROWS, S_ROWS)], srcbuf, sem_in),
        pltpu.async_copy(ei4.at[1, pl.ds(w * S_ROWS, S_ROWS)], dstbuf, sem_in),
    ]

    zv = jnp.zeros((16,), jnp.float32)
    ov = zv + 1.0

    def fill_zero(j, carry):
        for u in range(UNROLL):
            zerobuf[pl.ds((j * UNROLL + u) * 16, 16)] = zv
        return carry

    lax.fori_loop(0, N // 16 // UNROLL, fill_zero, None)

    def fill_one(j, carry):
        onesrow[pl.ds(j * 16, 16)] = ov
        return carry

    lax.fori_loop(0, CH // 16, fill_one, None)

    zs = [
        pltpu.async_copy(zerobuf.at[pl.ds(0, CHUNK)],
                         deg_sh.at[pl.ds(t * CHUNK, CHUNK)], sem_z),
        pltpu.async_copy(zerobuf, s_sh.at[pl.ds(t * N, N)], sem_z),
    ]
    for dsc in loads:
        dsc.wait()
    for dsc in zs:
        dsc.wait()
    plsc.subcore_barrier()

    deg_descs = [
        pltpu.async_copy(onesrow, deg_sh.at[degbuf.at[j, 0]], sem_sc, add=True)
        for j in range(DEG_ROWS)
    ]

    for r in range(S_ROWS):
        def idx_step(jo, carry):
            for u in range(UNROLL):
                k = jo * UNROLL + u
                sl = pl.ds(k * 16, 16)
                sv = srcbuf[r, 0, sl]
                dv = dstbuf[r, 0, sl]
                g = plsc.load_gather(i_priv, [dv])
                idxbuf[r, 0, sl] = g * N + sv
            return carry
        lax.fori_loop(0, VR // UNROLL, idx_step, None)

    @pl.when(c == 0)
    def _self_idx():
        def self_idx_step(k, carry):
            sl = pl.ds(k * 16, 16)
            v = t * CHUNK + k * 16 + lax.iota(jnp.int32, 16)
            valid = v < N
            vc = jnp.minimum(v, N - 1)
            g = plsc.load_gather(i_priv, [vc])
            selfidx[0, 0, sl] = jnp.where(valid, g * N + vc, 0)
            return carry
        lax.fori_loop(0, SELF_VR, self_idx_step, None)

    for dsc in deg_descs:
        dsc.wait()
    plsc.subcore_barrier()

    pltpu.sync_copy(deg_sh.at[pl.ds(t * CHUNK, CHUNK)], workbuf)

    def dinv_step(j, carry):
        d = workbuf[pl.ds(j * 16, 16)] + 1.0
        workbuf[pl.ds(j * 16, 16)] = _rsqrt_sc(d)
        return carry

    lax.fori_loop(0, CHUNK // 16, dinv_step, None)
    pltpu.sync_copy(workbuf, dinv_sh.at[pl.ds(t * CHUNK, CHUNK)])
    plsc.subcore_barrier()

    pltpu.sync_copy(dinv_sh, dinv_priv)

    for r in range(S_ROWS):
        def val_step(jo, carry):
            for u in range(UNROLL):
                k = jo * UNROLL + u
                sl = pl.ds(k * 16, 16)
                sv = srcbuf[r, 0, sl]
                dv = dstbuf[r, 0, sl]
                da = plsc.load_gather(dinv_priv, [sv])
                db = plsc.load_gather(dinv_priv, [dv])
                valbuf[r, 0, sl] = da * db
            return carry
        lax.fori_loop(0, VR // UNROLL, val_step, None)

    s_descs = [
        pltpu.async_copy(valbuf.at[j, 0], s_sh.at[idxbuf.at[j, 0]], sem_sc,
                         add=True)
        for j in range(S_ROWS)
    ]

    @pl.when(c == 0)
    def _self_loops():
        def self_val_step(k, carry):
            sl = pl.ds(k * 16, 16)
            v = t * CHUNK + k * 16 + lax.iota(jnp.int32, 16)
            valid = v < N
            vc = jnp.minimum(v, N - 1)
            dv = plsc.load_gather(dinv_priv, [vc])
            selfval[0, 0, sl] = jnp.where(valid, dv * dv, 0.0)
            return carry

        lax.fori_loop(0, SELF_VR, self_val_step, None)
        pltpu.async_copy(selfval.at[0, 0], s_sh.at[selfidx.at[0, 0]], sem_sc,
                         add=True).wait()

    for dsc in s_descs:
        dsc.wait()
    plsc.subcore_barrier()

    pltpu.sync_copy(s_sh.at[pl.ds(t * N, N)], zerobuf)
    pltpu.sync_copy(zerobuf, out_s.at[pl.ds(w * N, N)])


_sc_scatter = functools.partial(
    pl.kernel,
    out_type=jax.ShapeDtypeStruct((NC * NT * N,), jnp.float32),
    mesh=plsc.VectorSubcoreMesh(core_axis_name="c", subcore_axis_name="s"),
    compiler_params=pltpu.CompilerParams(needs_layout_passes=False),
    scratch_types=[
        pltpu.VMEM_SHARED((NPAD,), jnp.float32),
        pltpu.VMEM_SHARED((NPAD,), jnp.float32),
        pltpu.VMEM_SHARED((N * G,), jnp.float32),
        pltpu.VMEM((N,), jnp.float32),
        pltpu.VMEM((CH,), jnp.float32),
        pltpu.VMEM((DEG_ROWS, 1, CH), jnp.int32),
        pltpu.VMEM((S_ROWS, 1, CH), jnp.int32),
        pltpu.VMEM((S_ROWS, 1, CH), jnp.int32),
        pltpu.VMEM((N,), jnp.int32),
        pltpu.VMEM((NPAD,), jnp.float32),
        pltpu.VMEM((CHUNK,), jnp.float32),
        pltpu.VMEM((S_ROWS, 1, CH), jnp.int32),
        pltpu.VMEM((S_ROWS, 1, CH), jnp.float32),
        pltpu.VMEM((1, 1, CHUNK), jnp.int32),
        pltpu.VMEM((1, 1, CHUNK), jnp.float32),
        pltpu.SemaphoreType.DMA,
        pltpu.SemaphoreType.DMA,
        pltpu.SemaphoreType.DMA,
    ],
)(_sc_body)


def _tc_body(s_ref, x_ref, i_ref, w_ref, b_ref, wd_ref, bd_ref, o_ref):
    S = s_ref[0] + s_ref[1]
    X = x_ref[...]
    H = jnp.dot(X, w_ref[...])
    P = jnp.dot(S, H, precision=lax.Precision.HIGHEST)
    giota = lax.broadcasted_iota(jnp.int32, (N, G), 1)
    onehot = jnp.where(i_ref[...] == giota, 1.0, 0.0)
    ncol = lax.dot_general(onehot, jnp.ones((N, 1), jnp.float32),
                           (((0,), (0,)), ((), ())),
                           precision=lax.Precision.HIGHEST)
    pooled = P + ncol * b_ref[...]
    logits = jnp.dot(pooled, wd_ref[...],
                     precision=lax.Precision.HIGHEST) + bd_ref[...]
    m = jnp.max(logits, axis=1, keepdims=True)
    e = jnp.exp(logits - m)
    o_ref[...] = e / jnp.sum(e, axis=1, keepdims=True)


def kernel(x, edge_index, i, W, b, Wd, bd):
    ei4 = edge_index.astype(jnp.int32).reshape(2, E // CH, 1, CH)
    ii = i.astype(jnp.int32)

    s_flat = _sc_scatter(ei4, ii)
    s2 = s_flat.reshape(NC, G, N)

    out = pl.pallas_call(
        _tc_body,
        out_shape=jax.ShapeDtypeStruct((G, 10), jnp.float32),
    )(s2, x, ii.reshape(N, 1), W, b.reshape(1, D), Wd, bd.reshape(1, 10))
    return out
